# Initial kernel scaffold; baseline (speedup 1.0000x reference)
import jax
import jax.numpy as jnp
from jax.experimental import pallas as pl


def kernel(x_user, x_item, edge_index_user_item, edge_index_item_user, params):
    raise NotImplementedError("write your pallas kernel here")


# speedup vs baseline: 6.6119x; 6.6119x over previous
import functools

import jax
import jax.numpy as jnp
from jax import lax
from jax.experimental import pallas as pl
from jax.experimental.pallas import tpu as pltpu
from jax.experimental.pallas import tpu_sc as plsc

ALPHA = 0.15
K_ITERS = 10

N = 10000
NS = 10240
NTILES = 16
NPT = NS // NTILES
SLAB = 128
D = 64
CH = 128
GRP = 4
NCH = 160
NG = NCH // GRP
EPAD = NTILES * NCH * CH

_CHUNK_BYTES = CH * D * 4


def _mlp_body(x_ref, w1, b1, g1, be1, w2, b2, g2, be2, w3, b3, o_ref):
    def ln(h, g, b):
        mu = jnp.mean(h, axis=-1, keepdims=True)
        var = jnp.mean((h - mu) ** 2, axis=-1, keepdims=True)
        return (h - mu) / jnp.sqrt(var + 1e-5) * g[0] + b[0]

    def elu(h):
        return jnp.where(h > 0, h, jnp.exp(jnp.minimum(h, 0.0)) - 1.0)

    x = x_ref[0]
    h = jnp.dot(x, w1[0], preferred_element_type=jnp.float32) + b1[0]
    h = elu(ln(h, g1, be1))
    h = jnp.dot(h, w2[0], preferred_element_type=jnp.float32) + b2[0]
    h = elu(ln(h, g2, be2))
    o_ref[0] = jnp.dot(h, w3[0], preferred_element_type=jnp.float32) + b3[0]


def _mlp_fused(x_user, x_item, pu, pi):
    xs = jnp.stack([x_user, x_item])
    def st(name):
        a = jnp.stack([pu[name], pi[name]])
        return a.reshape(2, 1, -1) if a.ndim == 2 else a
    RB = 1000
    grid = (2, N // RB)
    vspec = lambda shp: pl.BlockSpec((1,) + shp, lambda i, j: (i,) + (0,) * len(shp))
    out = pl.pallas_call(
        _mlp_body,
        grid=grid,
        in_specs=[
            pl.BlockSpec((1, RB, 128), lambda i, j: (i, j, 0)),
            vspec((128, 128)), vspec((1, 128)), vspec((1, 128)), vspec((1, 128)),
            vspec((128, 128)), vspec((1, 128)), vspec((1, 128)), vspec((1, 128)),
            vspec((128, D)), vspec((1, D)),
        ],
        out_specs=pl.BlockSpec((1, RB, D), lambda i, j: (i, j, 0)),
        out_shape=jax.ShapeDtypeStruct((2, N, D), jnp.float32),
    )(xs, st('W1'), st('b1'), st('g1'), st('be1'),
      st('W2'), st('b2'), st('g2'), st('be2'), st('W3'), st('b3'))
    return out.reshape(2 * N, D)


_MESH = dict(core_axis_name="c", subcore_axis_name="s")


def _fill(buf, rows, value):
    v = jnp.full((16,), value, jnp.float32)

    def body(r, _):
        for col in range(0, D, 16):
            buf[r, pl.ds(col, 16)] = v
        return 0

    lax.fori_loop(0, rows, body, 0)


def _zero_acc(accbuf, acc_sh, s):
    _fill(accbuf, SLAB, 0.0)
    base = s * NPT
    for jj in range(5):
        pltpu.sync_copy(accbuf, acc_sh.at[pl.ds(base + jj * SLAB, SLAB)])


def _dwait(hbm_ref, vbuf, sem):
    pltpu.make_async_copy(hbm_ref.at[pl.ds(0, CH)], vbuf, sem).wait()


def _deg_body(dst_ref, r_ref, didx, onesbuf, accbuf, acc_sh, sem):
    c = lax.axis_index("c")
    s = lax.axis_index("s")
    w = c * NTILES + s
    side = 1 - c

    pltpu.sync_copy(dst_ref.at[w], didx)
    _fill(onesbuf, CH, 1.0)
    _zero_acc(accbuf, acc_sh, s)
    plsc.subcore_barrier()

    def body(j, _):
        pltpu.async_copy(onesbuf, acc_sh.at[didx.at[j]], sem, add=True)

        @pl.when(j >= GRP - 1)
        def _():
            _dwait(r_ref, onesbuf, sem)
        return 0

    lax.fori_loop(0, NCH, body, 0)
    for _ in range(GRP - 1):
        _dwait(r_ref, onesbuf, sem)
    plsc.subcore_barrier()

    base = s * NPT
    out_base = side * NS + base
    for jj in range(5):
        ofs = jj * SLAB
        pltpu.sync_copy(acc_sh.at[pl.ds(base + ofs, SLAB)], accbuf)

        def nrow(r, _):
            for col in range(0, D, 16):
                d = pl.ds(col, 16)
                accbuf[r, d] = (1.0 - ALPHA) / jnp.maximum(accbuf[r, d], 1.0)
            return 0

        lax.fori_loop(0, SLAB, nrow, 0)
        pltpu.sync_copy(accbuf, r_ref.at[pl.ds(out_base + ofs, SLAB)])


def _deg_call(dsts):
    return pl.kernel(
        _deg_body,
        out_type=jax.ShapeDtypeStruct((2 * NS, D), jnp.float32),
        mesh=plsc.VectorSubcoreMesh(**_MESH),
        compiler_params=pltpu.CompilerParams(use_tc_tiling_on_sc=False),
        scratch_types=[
            pltpu.VMEM((NCH, CH), jnp.int32),
            pltpu.VMEM((CH, D), jnp.float32),
            pltpu.VMEM((SLAB, D), jnp.float32),
            pltpu.VMEM_SHARED((NS, D), jnp.float32),
            pltpu.SemaphoreType.DMA,
        ],
    )(dsts)


def _prop_body(h_ref, src_ref, dst_ref, r_ref, h0_ref, out_ref,
               sidx, didx, bufs, acc_sh,
               g0, g1, g2, g3, s0, s1, s2, s3):
    gsems = (g0, g1, g2, g3)
    ssems = (s0, s1, s2, s3)
    c = lax.axis_index("c")
    s = lax.axis_index("s")
    w = c * NTILES + s
    side = 1 - c

    pltpu.sync_copy(src_ref.at[w], sidx)
    pltpu.sync_copy(dst_ref.at[w], didx)
    _zero_acc(bufs.at[0], acc_sh, s)
    plsc.subcore_barrier()

    for b in range(GRP):
        pltpu.async_copy(h_ref.at[sidx.at[b]], bufs.at[b], gsems[b])

    def grp(j, _):
        for b in range(GRP):
            k = j * GRP + b
            _dwait(h_ref, bufs.at[b], gsems[b])
            pltpu.async_copy(bufs.at[b], acc_sh.at[didx.at[k]],
                             ssems[b], add=True)
        for b in range(GRP):
            _dwait(h_ref, bufs.at[b], ssems[b])

            @pl.when(j < NG - 1)
            def _():
                kk = (j + 1) * GRP + b
                pltpu.async_copy(h_ref.at[sidx.at[kk]], bufs.at[b], gsems[b])
        return 0

    lax.fori_loop(0, NG, grp, 0)
    plsc.subcore_barrier()

    base = s * NPT
    out_base = side * NS + base
    accbuf = bufs.at[0]
    rbuf = bufs.at[1]
    h0buf = bufs.at[2]
    for jj in range(5):
        ofs = jj * SLAB
        pltpu.sync_copy(acc_sh.at[pl.ds(base + ofs, SLAB)], accbuf)
        pltpu.sync_copy(r_ref.at[pl.ds(out_base + ofs, SLAB)], rbuf)
        pltpu.sync_copy(h0_ref.at[pl.ds(out_base + ofs, SLAB)], h0buf)

        def nrow(r, _):
            for col in range(0, D, 16):
                d = pl.ds(col, 16)
                accbuf[r, d] = (accbuf[r, d] * rbuf[r, d]
                                + ALPHA * h0buf[r, d])
            return 0

        lax.fori_loop(0, SLAB, nrow, 0)
        pltpu.sync_copy(accbuf, out_ref.at[pl.ds(out_base + ofs, SLAB)])


def _prop_call(h, srcs, dsts, R, h0):
    return pl.kernel(
        _prop_body,
        out_type=jax.ShapeDtypeStruct((2 * NS, D), jnp.float32),
        mesh=plsc.VectorSubcoreMesh(**_MESH),
        compiler_params=pltpu.CompilerParams(use_tc_tiling_on_sc=False),
        scratch_types=[
            pltpu.VMEM((NCH, CH), jnp.int32),
            pltpu.VMEM((NCH, CH), jnp.int32),
            pltpu.VMEM((GRP, CH, D), jnp.float32),
            pltpu.VMEM_SHARED((NS, D), jnp.float32),
        ] + [pltpu.SemaphoreType.DMA] * 8,
    )(h, srcs, dsts, R, h0)


def _prep_edges(ei_ui, ei_iu):
    E = ei_ui.shape[1]
    pad = EPAD - E

    def prep(src, dst, table_base):
        src = src.astype(jnp.int32) + table_base
        dst = dst.astype(jnp.int32)
        src = jnp.concatenate([src, jnp.full((pad,), table_base, jnp.int32)])
        dst = jnp.concatenate([dst, jnp.full((pad,), N, jnp.int32)])
        return src.reshape(NTILES, NCH, CH), dst.reshape(NTILES, NCH, CH)

    s0, d0 = prep(ei_ui[0], ei_ui[1], 0)
    s1, d1 = prep(ei_iu[0], ei_iu[1], NS)
    srcs = jnp.concatenate([s0[None], s1[None]]).reshape(2 * NTILES, NCH, CH)
    dsts = jnp.concatenate([d0[None], d1[None]]).reshape(2 * NTILES, NCH, CH)
    return srcs, dsts


def kernel(x_user, x_item, edge_index_user_item, edge_index_item_user, params):
    h0 = _mlp_fused(x_user, x_item, params['user'], params['item'])
    h0 = jnp.pad(h0.reshape(2, N, D), ((0, 0), (0, NS - N), (0, 0)))
    h0 = h0.reshape(2 * NS, D)
    srcs, dsts = _prep_edges(edge_index_user_item, edge_index_item_user)
    R = _deg_call(dsts)
    h = h0
    for _ in range(K_ITERS):
        h = _prop_call(h, srcs, dsts, R, h0)
    return h[:N], h[NS:NS + N]


# speedup vs baseline: 6.6455x; 1.0051x over previous
import functools

import jax
import jax.numpy as jnp
from jax import lax
from jax.experimental import pallas as pl
from jax.experimental.pallas import tpu as pltpu
from jax.experimental.pallas import tpu_sc as plsc

ALPHA = 0.15
K_ITERS = 10

N = 10000
NS = 10240
NTILES = 16
NPT = NS // NTILES
SLAB = 128
D = 64
CH = 128
GRP = 5
NCH = 160
NG = NCH // GRP
EPAD = NTILES * NCH * CH

_CHUNK_BYTES = CH * D * 4


def _mlp_body(x_ref, w1, b1, g1, be1, w2, b2, g2, be2, w3, b3, o_ref):
    def ln(h, g, b):
        mu = jnp.mean(h, axis=-1, keepdims=True)
        var = jnp.mean((h - mu) ** 2, axis=-1, keepdims=True)
        return (h - mu) / jnp.sqrt(var + 1e-5) * g[0] + b[0]

    def elu(h):
        return jnp.where(h > 0, h, jnp.exp(jnp.minimum(h, 0.0)) - 1.0)

    x = x_ref[0]
    h = jnp.dot(x, w1[0], preferred_element_type=jnp.float32) + b1[0]
    h = elu(ln(h, g1, be1))
    h = jnp.dot(h, w2[0], preferred_element_type=jnp.float32) + b2[0]
    h = elu(ln(h, g2, be2))
    o_ref[0] = jnp.dot(h, w3[0], preferred_element_type=jnp.float32) + b3[0]


def _mlp_fused(x_user, x_item, pu, pi):
    xs = jnp.stack([x_user, x_item])
    def st(name):
        a = jnp.stack([pu[name], pi[name]])
        return a.reshape(2, 1, -1) if a.ndim == 2 else a
    RB = 1000
    grid = (2, N // RB)
    vspec = lambda shp: pl.BlockSpec((1,) + shp, lambda i, j: (i,) + (0,) * len(shp))
    out = pl.pallas_call(
        _mlp_body,
        grid=grid,
        in_specs=[
            pl.BlockSpec((1, RB, 128), lambda i, j: (i, j, 0)),
            vspec((128, 128)), vspec((1, 128)), vspec((1, 128)), vspec((1, 128)),
            vspec((128, 128)), vspec((1, 128)), vspec((1, 128)), vspec((1, 128)),
            vspec((128, D)), vspec((1, D)),
        ],
        out_specs=pl.BlockSpec((1, RB, D), lambda i, j: (i, j, 0)),
        out_shape=jax.ShapeDtypeStruct((2, N, D), jnp.float32),
    )(xs, st('W1'), st('b1'), st('g1'), st('be1'),
      st('W2'), st('b2'), st('g2'), st('be2'), st('W3'), st('b3'))
    return out.reshape(2 * N, D)


_MESH = dict(core_axis_name="c", subcore_axis_name="s")


def _fill(buf, rows, value):
    v = jnp.full((16,), value, jnp.float32)

    def body(r, _):
        for col in range(0, D, 16):
            buf[r, pl.ds(col, 16)] = v
        return 0

    lax.fori_loop(0, rows, body, 0)


def _zero_acc(accbuf, acc_sh, s):
    _fill(accbuf, SLAB, 0.0)
    base = s * NPT
    for jj in range(5):
        pltpu.sync_copy(accbuf, acc_sh.at[pl.ds(base + jj * SLAB, SLAB)])


def _dwait(hbm_ref, vbuf, sem):
    pltpu.make_async_copy(hbm_ref.at[pl.ds(0, CH)], vbuf, sem).wait()


def _deg_body(dst_ref, r_ref, didx, onesbuf, accbuf, acc_sh, sem):
    c = lax.axis_index("c")
    s = lax.axis_index("s")
    w = c * NTILES + s
    side = 1 - c

    pltpu.sync_copy(dst_ref.at[w], didx)
    _fill(onesbuf, CH, 1.0)
    _zero_acc(accbuf, acc_sh, s)
    plsc.subcore_barrier()

    def body(j, _):
        pltpu.async_copy(onesbuf, acc_sh.at[didx.at[j]], sem, add=True)

        @pl.when(j >= GRP - 1)
        def _():
            _dwait(r_ref, onesbuf, sem)
        return 0

    lax.fori_loop(0, NCH, body, 0)
    for _ in range(GRP - 1):
        _dwait(r_ref, onesbuf, sem)
    plsc.subcore_barrier()

    base = s * NPT
    out_base = side * NS + base
    for jj in range(5):
        ofs = jj * SLAB
        pltpu.sync_copy(acc_sh.at[pl.ds(base + ofs, SLAB)], accbuf)

        def nrow(r, _):
            for col in range(0, D, 16):
                d = pl.ds(col, 16)
                accbuf[r, d] = (1.0 - ALPHA) / jnp.maximum(accbuf[r, d], 1.0)
            return 0

        lax.fori_loop(0, SLAB, nrow, 0)
        pltpu.sync_copy(accbuf, r_ref.at[pl.ds(out_base + ofs, SLAB)])


def _deg_call(dsts):
    return pl.kernel(
        _deg_body,
        out_type=jax.ShapeDtypeStruct((2 * NS, D), jnp.float32),
        mesh=plsc.VectorSubcoreMesh(**_MESH),
        compiler_params=pltpu.CompilerParams(use_tc_tiling_on_sc=False),
        scratch_types=[
            pltpu.VMEM((NCH, CH), jnp.int32),
            pltpu.VMEM((CH, D), jnp.float32),
            pltpu.VMEM((SLAB, D), jnp.float32),
            pltpu.VMEM_SHARED((NS, D), jnp.float32),
            pltpu.SemaphoreType.DMA,
        ],
    )(dsts)


def _prop_body(h_ref, src_ref, dst_ref, r_ref, h0_ref, out_ref,
               sidx, didx, bufs, acc_sh,
               g0, g1, g2, g3, g4, s0, s1, s2, s3, s4):
    gsems = (g0, g1, g2, g3, g4)
    ssems = (s0, s1, s2, s3, s4)
    c = lax.axis_index("c")
    s = lax.axis_index("s")
    w = c * NTILES + s
    side = 1 - c

    pltpu.sync_copy(src_ref.at[w], sidx)
    pltpu.sync_copy(dst_ref.at[w], didx)
    _zero_acc(bufs.at[0], acc_sh, s)
    plsc.subcore_barrier()

    for b in range(GRP):
        pltpu.async_copy(h_ref.at[sidx.at[b]], bufs.at[b], gsems[b])

    def grp(j, _):
        for b in range(GRP):
            k = j * GRP + b
            _dwait(h_ref, bufs.at[b], gsems[b])
            pltpu.async_copy(bufs.at[b], acc_sh.at[didx.at[k]],
                             ssems[b], add=True)
        for b in range(GRP):
            _dwait(h_ref, bufs.at[b], ssems[b])

            @pl.when(j < NG - 1)
            def _():
                kk = (j + 1) * GRP + b
                pltpu.async_copy(h_ref.at[sidx.at[kk]], bufs.at[b], gsems[b])
        return 0

    lax.fori_loop(0, NG, grp, 0)
    plsc.subcore_barrier()

    base = s * NPT
    out_base = side * NS + base
    accbuf = bufs.at[0]
    rbuf = bufs.at[1]
    h0buf = bufs.at[2]
    for jj in range(5):
        ofs = jj * SLAB
        pltpu.sync_copy(acc_sh.at[pl.ds(base + ofs, SLAB)], accbuf)
        pltpu.sync_copy(r_ref.at[pl.ds(out_base + ofs, SLAB)], rbuf)
        pltpu.sync_copy(h0_ref.at[pl.ds(out_base + ofs, SLAB)], h0buf)

        def nrow(r, _):
            for col in range(0, D, 16):
                d = pl.ds(col, 16)
                accbuf[r, d] = (accbuf[r, d] * rbuf[r, d]
                                + ALPHA * h0buf[r, d])
            return 0

        lax.fori_loop(0, SLAB, nrow, 0)
        pltpu.sync_copy(accbuf, out_ref.at[pl.ds(out_base + ofs, SLAB)])


def _prop_call(h, srcs, dsts, R, h0):
    return pl.kernel(
        _prop_body,
        out_type=jax.ShapeDtypeStruct((2 * NS, D), jnp.float32),
        mesh=plsc.VectorSubcoreMesh(**_MESH),
        compiler_params=pltpu.CompilerParams(use_tc_tiling_on_sc=False),
        scratch_types=[
            pltpu.VMEM((NCH, CH), jnp.int32),
            pltpu.VMEM((NCH, CH), jnp.int32),
            pltpu.VMEM((GRP, CH, D), jnp.float32),
            pltpu.VMEM_SHARED((NS, D), jnp.float32),
        ] + [pltpu.SemaphoreType.DMA] * 10,
    )(h, srcs, dsts, R, h0)


def _prep_edges(ei_ui, ei_iu):
    E = ei_ui.shape[1]
    pad = EPAD - E

    def prep(src, dst, table_base):
        src = src.astype(jnp.int32) + table_base
        dst = dst.astype(jnp.int32)
        src = jnp.concatenate([src, jnp.full((pad,), table_base, jnp.int32)])
        dst = jnp.concatenate([dst, jnp.full((pad,), N, jnp.int32)])
        return src.reshape(NTILES, NCH, CH), dst.reshape(NTILES, NCH, CH)

    s0, d0 = prep(ei_ui[0], ei_ui[1], 0)
    s1, d1 = prep(ei_iu[0], ei_iu[1], NS)
    srcs = jnp.concatenate([s0[None], s1[None]]).reshape(2 * NTILES, NCH, CH)
    dsts = jnp.concatenate([d0[None], d1[None]]).reshape(2 * NTILES, NCH, CH)
    return srcs, dsts


def kernel(x_user, x_item, edge_index_user_item, edge_index_item_user, params):
    h0 = _mlp_fused(x_user, x_item, params['user'], params['item'])
    h0 = jnp.pad(h0.reshape(2, N, D), ((0, 0), (0, NS - N), (0, 0)))
    h0 = h0.reshape(2 * NS, D)
    srcs, dsts = _prep_edges(edge_index_user_item, edge_index_item_user)
    R = _deg_call(dsts)
    h = h0
    for _ in range(K_ITERS):
        h = _prop_call(h, srcs, dsts, R, h0)
    return h[:N], h[NS:NS + N]
